# baseline (speedup 1.0000x reference)
import jax
import jax.numpy as jnp
import numpy as np
from jax.experimental import pallas as pl
from jax.experimental.pallas import tpu as pltpu

DEPTH = 5
HID = 128
NUM_GRAPHS = 64


def _lin(p, x):
    return x @ p["W"] + p["b"]


def _ln(p, x):
    mu = x.mean(-1, keepdims=True)
    var = x.var(-1, keepdims=True)
    return (x - mu) / jnp.sqrt(var + 1e-5) * p["g"] + p["b"]


def _egnn_layer(p, h, pos, edge_index):
    src, dst = edge_index[0], edge_index[1]
    d = jnp.linalg.norm(pos[dst] - pos[src], axis=-1, keepdims=True)
    m = jnp.concatenate([h[dst], h[src], d], axis=-1)
    m = jax.nn.silu(_ln(p["ln1"], _lin(p["msg1"], m)))
    m = jax.nn.silu(_ln(p["ln2"], _lin(p["msg2"], m)))
    agg = jax.ops.segment_sum(m, dst, num_segments=h.shape[0])
    u = jnp.concatenate([h, agg], axis=-1)
    u = jax.nn.silu(_ln(p["ln3"], _lin(p["upd1"], u)))
    u = jax.nn.silu(_ln(p["ln4"], _lin(p["upd2"], u)))
    return u


def _tconv(p, h, edge_index):
    n = h.shape[0]
    src, dst = edge_index[0], edge_index[1]
    C = HID
    q = _lin(p["q"], h)
    k = _lin(p["k"], h)
    v = _lin(p["v"], h)
    logits = (q[dst] * k[src]).sum(-1) / np.sqrt(C)
    mx = jax.ops.segment_max(logits, dst, num_segments=n)
    ex = jnp.exp(logits - mx[dst])
    denom = jax.ops.segment_sum(ex, dst, num_segments=n)
    alpha = ex / (denom[dst] + 1e-16)
    out = jax.ops.segment_sum(v[src] * alpha[:, None], dst, num_segments=n)
    return out + _lin(p["skip"], h)


def _pred_kernel(pooled_ref, w1_ref, b1_ref, w2_ref, b2_ref, out_ref):
    t = jnp.maximum(pooled_ref[...] @ w1_ref[...] + b1_ref[...], 0.0)
    out_ref[...] = t @ w2_ref[...] + b2_ref[...]


def kernel(x, pos, edge_index, node_subnode_index, subgraph_edge_index, subnode_node_index, batch, params):
    h = _lin(params["emb_in"], x)
    for lp in params["layers"]:
        h0 = h
        h = _egnn_layer(lp["egnn"], h, pos, edge_index)
        h = _tconv(lp["g2s"], h, node_subnode_index)
        h = _tconv(lp["sub"], h, subgraph_edge_index)
        h = _tconv(lp["s2g"], h, subnode_node_index)
        h = h + h0
    pooled = jax.ops.segment_sum(h, batch, num_segments=NUM_GRAPHS)
    out = pl.pallas_call(
        _pred_kernel,
        out_shape=jax.ShapeDtypeStruct((NUM_GRAPHS, 1), jnp.float32),
    )(pooled, params["pred1"]["W"], params["pred1"]["b"][None, :],
      params["pred2"]["W"], params["pred2"]["b"][None, :])
    return out


# speedup vs baseline: 5.5231x; 5.5231x over previous
import functools

import jax
import jax.numpy as jnp
import numpy as np
from jax import lax
from jax.experimental import pallas as pl
from jax.experimental.pallas import tpu as pltpu
from jax.experimental.pallas import tpu_sc as plsc

N = 10000
E = 320000
C = 128
NG = 64
NC = 2
NS = 16
L = 16
NW = NC * NS
EPW = E // NW
N2 = 10240
NPS = N2 // NS

CH_G = 80
CH_D = 80

_MESH = plsc.VectorSubcoreMesh(core_axis_name="c", subcore_axis_name="s")
_INV_SQRT_C = 1.0 / np.sqrt(np.float32(C))


def _gather16(tab_v, iv, lanes):
    out = jnp.zeros((L,), jnp.float32)
    for j in range(L):
        lv = tab_v[pl.ds(iv[j], L)]
        out = jnp.where(lanes == j, jnp.full((L,), lv[0], jnp.float32), out)
    return out


def _wid():
    return lax.axis_index("s") * NC + lax.axis_index("c")


@functools.partial(
    pl.kernel, mesh=_MESH,
    out_type=(jax.ShapeDtypeStruct((E, C), jnp.float32),
              jax.ShapeDtypeStruct((E, C), jnp.float32)),
    scratch_types=[
        pltpu.VMEM((EPW,), jnp.int32), pltpu.VMEM((EPW,), jnp.int32),
        pltpu.VMEM((CH_G, C), jnp.float32), pltpu.VMEM((CH_G, C), jnp.float32),
        pltpu.VMEM((CH_G, C), jnp.float32), pltpu.VMEM((CH_G, C), jnp.float32),
        pltpu.SemaphoreType.DMA, pltpu.SemaphoreType.DMA,
        pltpu.SemaphoreType.DMA, pltpu.SemaphoreType.DMA,
    ],
)
def _sc_gather2(t1_hbm, t2_hbm, dst_hbm, src_hbm, hd_hbm, hs_hbm,
                idxd_v, idxs_v, rdA_v, rsA_v, rdB_v, rsB_v,
                semA1, semA2, semB1, semB2):
    base = _wid() * EPW
    pltpu.sync_copy(dst_hbm.at[pl.ds(base, EPW)], idxd_v)
    pltpu.sync_copy(src_hbm.at[pl.ds(base, EPW)], idxs_v)
    nch = EPW // CH_G

    def body(p, carry):
        o0 = (2 * p) * CH_G
        o1 = o0 + CH_G
        g1 = pltpu.async_copy(t1_hbm.at[idxd_v.at[pl.ds(o0, CH_G)]], rdA_v, semA1)
        g2 = pltpu.async_copy(t2_hbm.at[idxs_v.at[pl.ds(o0, CH_G)]], rsA_v, semA2)
        g3 = pltpu.async_copy(t1_hbm.at[idxd_v.at[pl.ds(o1, CH_G)]], rdB_v, semB1)
        g4 = pltpu.async_copy(t2_hbm.at[idxs_v.at[pl.ds(o1, CH_G)]], rsB_v, semB2)
        g1.wait()
        g2.wait()
        pltpu.sync_copy(rdA_v, hd_hbm.at[pl.ds(base + o0, CH_G)])
        pltpu.sync_copy(rsA_v, hs_hbm.at[pl.ds(base + o0, CH_G)])
        g3.wait()
        g4.wait()
        pltpu.sync_copy(rdB_v, hd_hbm.at[pl.ds(base + o1, CH_G)])
        pltpu.sync_copy(rsB_v, hs_hbm.at[pl.ds(base + o1, CH_G)])
        return carry

    lax.fori_loop(0, nch // 2, body, 0)
    ot = (nch - 1) * CH_G
    g1 = pltpu.async_copy(t1_hbm.at[idxd_v.at[pl.ds(ot, CH_G)]], rdA_v, semA1)
    g2 = pltpu.async_copy(t2_hbm.at[idxs_v.at[pl.ds(ot, CH_G)]], rsA_v, semA2)
    g1.wait()
    g2.wait()
    pltpu.sync_copy(rdA_v, hd_hbm.at[pl.ds(base + ot, CH_G)])
    pltpu.sync_copy(rsA_v, hs_hbm.at[pl.ds(base + ot, CH_G)])


ZR = 8


def _zero_spmem_slice(ztile_v, acc_sh, sid):
    zv = jnp.zeros((L,), jnp.float32)
    for r in range(ZR):
        for c in range(C // L):
            ztile_v[r, pl.ds(c * L, L)] = zv

    def cp(i, carry):
        pltpu.sync_copy(ztile_v, acc_sh.at[pl.ds(sid * NPS + i * ZR, ZR)])
        return carry

    lax.fori_loop(0, NPS // ZR, cp, 0)


@functools.partial(
    pl.kernel, mesh=_MESH,
    out_type=jax.ShapeDtypeStruct((NC, N2, C), jnp.float32),
    scratch_types=[
        pltpu.VMEM((CH_G,), jnp.int32), pltpu.VMEM((CH_G,), jnp.int32),
        pltpu.VMEM((CH_G, C), jnp.float32), pltpu.VMEM((CH_G, C), jnp.float32),
        pltpu.VMEM((ZR, C), jnp.float32),
        pltpu.VMEM_SHARED((N2, C), jnp.float32),
        pltpu.SemaphoreType.DMA, pltpu.SemaphoreType.DMA,
        pltpu.SemaphoreType.DMA, pltpu.SemaphoreType.DMA,
    ],
)
def _sc_scatter_rows(rows_hbm, dst_hbm, out_hbm, idxA_v, idxB_v, rowsA_v, rowsB_v,
                     ztile_v, acc_sh, semA1, semA2, semB1, semB2):
    cid = lax.axis_index("c")
    sid = lax.axis_index("s")
    base = _wid() * EPW
    _zero_spmem_slice(ztile_v, acc_sh, sid)
    plsc.subcore_barrier()
    nch = EPW // CH_G

    def body(p, carry):
        o0 = base + (2 * p) * CH_G
        o1 = o0 + CH_G
        lA1 = pltpu.async_copy(dst_hbm.at[pl.ds(o0, CH_G)], idxA_v, semA1)
        lA2 = pltpu.async_copy(rows_hbm.at[pl.ds(o0, CH_G)], rowsA_v, semA2)
        lB1 = pltpu.async_copy(dst_hbm.at[pl.ds(o1, CH_G)], idxB_v, semB1)
        lB2 = pltpu.async_copy(rows_hbm.at[pl.ds(o1, CH_G)], rowsB_v, semB2)
        lA1.wait()
        lA2.wait()
        pltpu.sync_copy(rowsA_v, acc_sh.at[idxA_v], add=True)
        lB1.wait()
        lB2.wait()
        pltpu.sync_copy(rowsB_v, acc_sh.at[idxB_v], add=True)
        return carry

    lax.fori_loop(0, nch // 2, body, 0)
    ot = base + (nch - 1) * CH_G
    pltpu.sync_copy(dst_hbm.at[pl.ds(ot, CH_G)], idxA_v)
    pltpu.sync_copy(rows_hbm.at[pl.ds(ot, CH_G)], rowsA_v)
    pltpu.sync_copy(rowsA_v, acc_sh.at[idxA_v], add=True)
    plsc.subcore_barrier()
    pltpu.sync_copy(acc_sh.at[pl.ds(sid * NPS, NPS)], out_hbm.at[cid, pl.ds(sid * NPS, NPS)])


@functools.partial(
    pl.kernel, mesh=_MESH,
    out_type=jax.ShapeDtypeStruct((NC, N2), jnp.float32),
    scratch_types=[
        pltpu.VMEM((EPW,), jnp.int32),
        pltpu.VMEM((EPW,), jnp.float32),
        pltpu.VMEM((N2,), jnp.float32),
        pltpu.VMEM((NPS,), jnp.float32), pltpu.VMEM((NPS,), jnp.float32),
        pltpu.VMEM_SHARED((NS, N2), jnp.float32),
    ],
)
def _sc_max(logits_hbm, dst_hbm, mxp_hbm,
            idxd_v, lg_v, bins_v, red_v, tmp_v, sh_v):
    cid = lax.axis_index("c")
    sid = lax.axis_index("s")
    base = _wid() * EPW

    neg = jnp.full((L,), -1e30, jnp.float32)

    def initbins(i, carry):
        bins_v[pl.ds(i * L, L)] = neg
        return carry

    lax.fori_loop(0, N2 // L, initbins, 0)
    pltpu.sync_copy(dst_hbm.at[pl.ds(base, EPW)], idxd_v)
    pltpu.sync_copy(logits_hbm.at[pl.ds(base, EPW)], lg_v)

    def group(g, carry2):
        iv = idxd_v[pl.ds(g * L, L)]
        acc = lg_v[pl.ds(g * L, L)]
        lane0 = jax.lax.iota(jnp.int32, L) == 0
        for j in range(L):
            ix = iv[j]
            sb = jnp.full((L,), acc[j], jnp.float32)
            lv = bins_v[pl.ds(ix, L)]
            bins_v[pl.ds(ix, L)] = jnp.where(lane0, jnp.maximum(lv, sb), lv)
        return carry2

    lax.fori_loop(0, EPW // L, group, 0)

    pltpu.sync_copy(bins_v, sh_v.at[sid])
    plsc.subcore_barrier()
    pltpu.sync_copy(sh_v.at[0, pl.ds(sid * NPS, NPS)], red_v)
    for t in range(1, NS):
        pltpu.sync_copy(sh_v.at[t, pl.ds(sid * NPS, NPS)], tmp_v)

        def mx(i, carry):
            red_v[pl.ds(i * L, L)] = jnp.maximum(red_v[pl.ds(i * L, L)], tmp_v[pl.ds(i * L, L)])
            return carry

        lax.fori_loop(0, NPS // L, mx, 0)
    pltpu.sync_copy(red_v, mxp_hbm.at[cid, pl.ds(sid * NPS, NPS)])


@functools.partial(
    pl.kernel, mesh=_MESH,
    out_type=(jax.ShapeDtypeStruct((E,), jnp.float32),
              jax.ShapeDtypeStruct((NC, N2), jnp.float32)),
    scratch_types=[
        pltpu.VMEM((EPW,), jnp.int32),
        pltpu.VMEM((EPW,), jnp.float32), pltpu.VMEM((EPW,), jnp.float32),
        pltpu.VMEM((N2,), jnp.float32), pltpu.VMEM((N2,), jnp.float32),
        pltpu.VMEM((N2,), jnp.float32),
        pltpu.VMEM((NPS,), jnp.float32), pltpu.VMEM((NPS,), jnp.float32),
        pltpu.VMEM_SHARED((NS, N2), jnp.float32),
    ],
)
def _sc_conv_ex(logits_hbm, dst_hbm, mxp_hbm, ex_hbm, denp_hbm,
                idxd_v, lg_v, ex_v, mxloc_v, mx2_v, bins_v, red_v, tmp_v, sh_v):
    cid = lax.axis_index("c")
    sid = lax.axis_index("s")
    base = _wid() * EPW

    pltpu.sync_copy(mxp_hbm.at[0], mxloc_v)
    pltpu.sync_copy(mxp_hbm.at[1], mx2_v)

    def cmb(i, carry):
        mxloc_v[pl.ds(i * L, L)] = jnp.maximum(mxloc_v[pl.ds(i * L, L)], mx2_v[pl.ds(i * L, L)])
        return carry

    lax.fori_loop(0, N2 // L, cmb, 0)

    def zb(i, carry):
        bins_v[pl.ds(i * L, L)] = jnp.zeros((L,), jnp.float32)
        return carry

    lax.fori_loop(0, N2 // L, zb, 0)

    pltpu.sync_copy(dst_hbm.at[pl.ds(base, EPW)], idxd_v)
    pltpu.sync_copy(logits_hbm.at[pl.ds(base, EPW)], lg_v)

    def group(g, carry2):
        iv = idxd_v[pl.ds(g * L, L)]
        lv = lg_v[pl.ds(g * L, L)]
        lanes0 = jax.lax.iota(jnp.int32, L)
        mxv = _gather16(mxloc_v, iv, lanes0)
        ev = jnp.exp(lv - mxv)
        ex_v[pl.ds(g * L, L)] = ev
        lane0 = jax.lax.iota(jnp.int32, L) == 0
        for j in range(L):
            ix = iv[j]
            sb = jnp.full((L,), ev[j], jnp.float32)
            cur = bins_v[pl.ds(ix, L)]
            bins_v[pl.ds(ix, L)] = jnp.where(lane0, cur + sb, cur)
        return carry2

    lax.fori_loop(0, EPW // L, group, 0)
    pltpu.sync_copy(ex_v, ex_hbm.at[pl.ds(base, EPW)])

    pltpu.sync_copy(bins_v, sh_v.at[sid])
    plsc.subcore_barrier()
    pltpu.sync_copy(sh_v.at[0, pl.ds(sid * NPS, NPS)], red_v)
    for t in range(1, NS):
        pltpu.sync_copy(sh_v.at[t, pl.ds(sid * NPS, NPS)], tmp_v)

        def ad(i, carry):
            red_v[pl.ds(i * L, L)] = red_v[pl.ds(i * L, L)] + tmp_v[pl.ds(i * L, L)]
            return carry

        lax.fori_loop(0, NPS // L, ad, 0)
    pltpu.sync_copy(red_v, denp_hbm.at[cid, pl.ds(sid * NPS, NPS)])


@functools.partial(
    pl.kernel, mesh=_MESH,
    out_type=jax.ShapeDtypeStruct((NC, N2, C), jnp.float32),
    scratch_types=[
        pltpu.VMEM((CH_D,), jnp.int32), pltpu.VMEM((CH_D,), jnp.int32),
        pltpu.VMEM((CH_D,), jnp.int32), pltpu.VMEM((CH_D,), jnp.int32),
        pltpu.VMEM((CH_D,), jnp.float32), pltpu.VMEM((CH_D,), jnp.float32),
        pltpu.VMEM((CH_D, C), jnp.float32), pltpu.VMEM((CH_D, C), jnp.float32),
        pltpu.VMEM((N2,), jnp.float32), pltpu.VMEM((N2,), jnp.float32),
        pltpu.VMEM((ZR, C), jnp.float32),
        pltpu.VMEM_SHARED((N2, C), jnp.float32),
        pltpu.SemaphoreType.DMA, pltpu.SemaphoreType.DMA,
        pltpu.SemaphoreType.DMA, pltpu.SemaphoreType.DMA,
    ],
)
def _sc_conv_scat(ex_hbm, denp_hbm, dst_hbm, src_hbm, v_hbm, out_hbm,
                  idxdA_v, idxdB_v, idxsA_v, idxsB_v, exA_v, exB_v, vrA_v, vrB_v,
                  denloc_v, tmp_v, ztile_v, acc_sh, semA, semB, semA2, semB2):
    cid = lax.axis_index("c")
    sid = lax.axis_index("s")
    base = _wid() * EPW

    pltpu.sync_copy(denp_hbm.at[0], denloc_v)
    pltpu.sync_copy(denp_hbm.at[1], tmp_v)

    def cmb(i, carry):
        denloc_v[pl.ds(i * L, L)] = denloc_v[pl.ds(i * L, L)] + tmp_v[pl.ds(i * L, L)]
        return carry

    lax.fori_loop(0, N2 // L, cmb, 0)

    _zero_spmem_slice(ztile_v, acc_sh, sid)
    plsc.subcore_barrier()

    def load(o, idxdC_v, idxsC_v, exC_v, vr_v, sem):
        pltpu.sync_copy(dst_hbm.at[pl.ds(o, CH_D)], idxdC_v)
        pltpu.sync_copy(src_hbm.at[pl.ds(o, CH_D)], idxsC_v)
        pltpu.sync_copy(ex_hbm.at[pl.ds(o, CH_D)], exC_v)
        return pltpu.async_copy(v_hbm.at[idxsC_v], vr_v, sem)

    def scale(idxdC_v, exC_v, vr_v):
        def group(g, carry2):
            iv = idxdC_v[pl.ds(g * L, L)]
            ev = exC_v[pl.ds(g * L, L)]
            lanes0 = jax.lax.iota(jnp.int32, L)
            dv = _gather16(denloc_v, iv, lanes0)
            av = ev / (dv + np.float32(1e-16))
            for j in range(L):
                e = g * L + j
                a = av[j]
                for c in range(C // L):
                    vr_v[e, pl.ds(c * L, L)] = vr_v[e, pl.ds(c * L, L)] * a
            return carry2

        lax.fori_loop(0, CH_D // L, group, 0)

    nch = EPW // CH_D

    def body(p, carry):
        o0 = base + (2 * p) * CH_D
        o1 = o0 + CH_D
        gA = load(o0, idxdA_v, idxsA_v, exA_v, vrA_v, semA)
        gB = load(o1, idxdB_v, idxsB_v, exB_v, vrB_v, semB)
        gA.wait()
        scale(idxdA_v, exA_v, vrA_v)
        pltpu.sync_copy(vrA_v, acc_sh.at[idxdA_v], add=True)
        gB.wait()
        scale(idxdB_v, exB_v, vrB_v)
        pltpu.sync_copy(vrB_v, acc_sh.at[idxdB_v], add=True)
        return carry

    lax.fori_loop(0, nch // 2, body, 0)
    ot = base + (nch - 1) * CH_D
    load(ot, idxdA_v, idxsA_v, exA_v, vrA_v, semA).wait()
    scale(idxdA_v, exA_v, vrA_v)
    pltpu.sync_copy(vrA_v, acc_sh.at[idxdA_v], add=True)
    plsc.subcore_barrier()
    pltpu.sync_copy(acc_sh.at[pl.ds(sid * NPS, NPS)], out_hbm.at[cid, pl.ds(sid * NPS, NPS)])


def _ln_rows(x, g, b):
    mu = jnp.mean(x, axis=-1, keepdims=True)
    var = jnp.mean((x - mu) * (x - mu), axis=-1, keepdims=True)
    return (x - mu) / jnp.sqrt(var + 1e-5) * g + b


def _silu(x):
    return x * jax.nn.sigmoid(x)


def _dot(a, b):
    return jax.lax.dot(a, b, precision=jax.lax.Precision.DEFAULT)


BE = 2000


def _tc_dcol_body(pd_ref, ps_ref, d_ref):
    diff = pd_ref[...] - ps_ref[...]
    ssq = jnp.sum(diff * diff, axis=-1, keepdims=True)
    d_ref[...] = jnp.sqrt(ssq)


def _tc_dcol(pd, ps):
    return pl.pallas_call(
        _tc_dcol_body,
        grid=(E // BE,),
        in_specs=[
            pl.BlockSpec((BE, C), lambda i: (i, 0)),
            pl.BlockSpec((BE, C), lambda i: (i, 0)),
        ],
        out_specs=pl.BlockSpec((BE, 1), lambda i: (i, 0)),
        out_shape=jax.ShapeDtypeStruct((E, 1), jnp.float32),
    )(pd, ps)


def _tc_dot_body(qd_ref, ks_ref, lg_ref):
    s = jnp.sum(qd_ref[...] * ks_ref[...], axis=-1, keepdims=True)
    lg_ref[...] = s / np.float32(np.sqrt(128.0))


def _tc_dot(qd, ks):
    return pl.pallas_call(
        _tc_dot_body,
        grid=(E // BE,),
        in_specs=[
            pl.BlockSpec((BE, C), lambda i: (i, 0)),
            pl.BlockSpec((BE, C), lambda i: (i, 0)),
        ],
        out_specs=pl.BlockSpec((BE, 1), lambda i: (i, 0)),
        out_shape=jax.ShapeDtypeStruct((E, 1), jnp.float32),
    )(qd, ks)


def _tc_edge_mlp_body(hd_ref, hs_ref, d_ref, w1_ref, b1_ref,
                      g1_ref, bb1_ref, w2_ref, b2_ref, g2_ref, bb2_ref, m_ref):
    d = d_ref[...]
    lane0 = (jax.lax.broadcasted_iota(jnp.int32, (BE, C), 1) == 0).astype(jnp.float32)
    dcol = d * lane0
    x = jnp.concatenate([hd_ref[...], hs_ref[...], dcol], axis=-1)
    m = _dot(x, w1_ref[...]) + b1_ref[...]
    m = _silu(_ln_rows(m, g1_ref[...], bb1_ref[...]))
    m = _dot(m, w2_ref[...]) + b2_ref[...]
    m = _silu(_ln_rows(m, g2_ref[...], bb2_ref[...]))
    m_ref[...] = m


def _tc_edge_mlp(hd, hs, d, p):
    w1 = jnp.concatenate([p["msg1"]["W"],
                          jnp.zeros((384 - 257, C), jnp.float32)], axis=0)
    row = lambda a: a[None, :]
    return pl.pallas_call(
        _tc_edge_mlp_body,
        grid=(E // BE,),
        in_specs=[
            pl.BlockSpec((BE, C), lambda i: (i, 0)),
            pl.BlockSpec((BE, C), lambda i: (i, 0)),
            pl.BlockSpec((BE, 1), lambda i: (i, 0)),
            pl.BlockSpec((384, C), lambda i: (0, 0)),
            pl.BlockSpec((1, C), lambda i: (0, 0)),
            pl.BlockSpec((1, C), lambda i: (0, 0)),
            pl.BlockSpec((1, C), lambda i: (0, 0)),
            pl.BlockSpec((C, C), lambda i: (0, 0)),
            pl.BlockSpec((1, C), lambda i: (0, 0)),
            pl.BlockSpec((1, C), lambda i: (0, 0)),
            pl.BlockSpec((1, C), lambda i: (0, 0)),
        ],
        out_specs=pl.BlockSpec((BE, C), lambda i: (i, 0)),
        out_shape=jax.ShapeDtypeStruct((E, C), jnp.float32),
    )(hd, hs, d, w1, row(p["msg1"]["b"]), row(p["ln1"]["g"]), row(p["ln1"]["b"]),
      p["msg2"]["W"], row(p["msg2"]["b"]), row(p["ln2"]["g"]), row(p["ln2"]["b"]))


BN = 2000


def _tc_egnn_node_body(h_ref, a0_ref, a1_ref, u1_ref, ub1_ref, g3_ref, b3_ref,
                       u2_ref, ub2_ref, g4_ref, b4_ref,
                       wq_ref, bq_ref, wk_ref, bk_ref, wv_ref, bv_ref,
                       h_out, q_out, k_out, v_out):
    agg = a0_ref[...] + a1_ref[...]
    x = jnp.concatenate([h_ref[...], agg], axis=-1)
    u = _dot(x, u1_ref[...]) + ub1_ref[...]
    u = _silu(_ln_rows(u, g3_ref[...], b3_ref[...]))
    u = _dot(u, u2_ref[...]) + ub2_ref[...]
    u = _silu(_ln_rows(u, g4_ref[...], b4_ref[...]))
    h_out[...] = u
    q_out[...] = _dot(u, wq_ref[...]) + bq_ref[...]
    k_out[...] = _dot(u, wk_ref[...]) + bk_ref[...]
    v_out[...] = _dot(u, wv_ref[...]) + bv_ref[...]


def _tc_egnn_node(h, aggp, pe, pc):
    row = lambda a: a[None, :]
    outs = (jax.ShapeDtypeStruct((N, C), jnp.float32),) * 4
    return pl.pallas_call(
        _tc_egnn_node_body,
        grid=(N // BN,),
        in_specs=[
            pl.BlockSpec((BN, C), lambda i: (i, 0)),
            pl.BlockSpec((BN, C), lambda i: (i, 0)),
            pl.BlockSpec((BN, C), lambda i: (i, 0)),
            pl.BlockSpec((2 * C, C), lambda i: (0, 0)),
            pl.BlockSpec((1, C), lambda i: (0, 0)),
            pl.BlockSpec((1, C), lambda i: (0, 0)),
            pl.BlockSpec((1, C), lambda i: (0, 0)),
            pl.BlockSpec((C, C), lambda i: (0, 0)),
            pl.BlockSpec((1, C), lambda i: (0, 0)),
            pl.BlockSpec((1, C), lambda i: (0, 0)),
            pl.BlockSpec((1, C), lambda i: (0, 0)),
            pl.BlockSpec((C, C), lambda i: (0, 0)),
            pl.BlockSpec((1, C), lambda i: (0, 0)),
            pl.BlockSpec((C, C), lambda i: (0, 0)),
            pl.BlockSpec((1, C), lambda i: (0, 0)),
            pl.BlockSpec((C, C), lambda i: (0, 0)),
            pl.BlockSpec((1, C), lambda i: (0, 0)),
        ],
        out_specs=[pl.BlockSpec((BN, C), lambda i: (i, 0))] * 4,
        out_shape=outs,
    )(h, aggp[0, :N], aggp[1, :N], pe["upd1"]["W"], row(pe["upd1"]["b"]),
      row(pe["ln3"]["g"]), row(pe["ln3"]["b"]), pe["upd2"]["W"], row(pe["upd2"]["b"]),
      row(pe["ln4"]["g"]), row(pe["ln4"]["b"]),
      pc["q"]["W"], row(pc["q"]["b"]), pc["k"]["W"], row(pc["k"]["b"]),
      pc["v"]["W"], row(pc["v"]["b"]))


def _tc_conv_fin_body(o0_ref, o1_ref, h_ref, ws_ref, bs_ref, h0_ref,
                      wq_ref, bq_ref, wk_ref, bk_ref, wv_ref, bv_ref,
                      h_out, q_out, k_out, v_out, *, residual, project):
    out = o0_ref[...] + o1_ref[...]
    hn = out + (_dot(h_ref[...], ws_ref[...]) + bs_ref[...])
    if residual:
        hn = hn + h0_ref[...]
    h_out[...] = hn
    if project:
        q_out[...] = _dot(hn, wq_ref[...]) + bq_ref[...]
        k_out[...] = _dot(hn, wk_ref[...]) + bk_ref[...]
        v_out[...] = _dot(hn, wv_ref[...]) + bv_ref[...]
    else:
        q_out[...] = jnp.zeros_like(hn)
        k_out[...] = jnp.zeros_like(hn)
        v_out[...] = jnp.zeros_like(hn)


def _tc_conv_fin(outp, h, pskip, h0, pc_next):
    row = lambda a: a[None, :]
    residual = h0 is not None
    project = pc_next is not None
    if h0 is None:
        h0 = h
    if pc_next is None:
        pc_next = {"q": pskip, "k": pskip, "v": pskip}
    outs = (jax.ShapeDtypeStruct((N, C), jnp.float32),) * 4
    body = functools.partial(_tc_conv_fin_body, residual=residual, project=project)
    return pl.pallas_call(
        body,
        grid=(N // BN,),
        in_specs=[
            pl.BlockSpec((BN, C), lambda i: (i, 0)),
            pl.BlockSpec((BN, C), lambda i: (i, 0)),
            pl.BlockSpec((BN, C), lambda i: (i, 0)),
            pl.BlockSpec((C, C), lambda i: (0, 0)),
            pl.BlockSpec((1, C), lambda i: (0, 0)),
            pl.BlockSpec((BN, C), lambda i: (i, 0)),
            pl.BlockSpec((C, C), lambda i: (0, 0)),
            pl.BlockSpec((1, C), lambda i: (0, 0)),
            pl.BlockSpec((C, C), lambda i: (0, 0)),
            pl.BlockSpec((1, C), lambda i: (0, 0)),
            pl.BlockSpec((C, C), lambda i: (0, 0)),
            pl.BlockSpec((1, C), lambda i: (0, 0)),
        ],
        out_specs=[pl.BlockSpec((BN, C), lambda i: (i, 0))] * 4,
        out_shape=outs,
    )(outp[0, :N], outp[1, :N], h, pskip["W"], row(pskip["b"]), h0,
      pc_next["q"]["W"], row(pc_next["q"]["b"]), pc_next["k"]["W"], row(pc_next["k"]["b"]),
      pc_next["v"]["W"], row(pc_next["v"]["b"]))


def _tc_emb_body(x_ref, w_ref, b_ref, h_ref):
    h_ref[...] = x_ref[...] * w_ref[...] + b_ref[...]


def _tc_emb(x, p):
    return pl.pallas_call(
        _tc_emb_body,
        grid=(N // BN,),
        in_specs=[
            pl.BlockSpec((BN, 1), lambda i: (i, 0)),
            pl.BlockSpec((1, C), lambda i: (0, 0)),
            pl.BlockSpec((1, C), lambda i: (0, 0)),
        ],
        out_specs=pl.BlockSpec((BN, C), lambda i: (i, 0)),
        out_shape=jax.ShapeDtypeStruct((N, C), jnp.float32),
    )(x, p["W"], p["b"][None, :])


def _tc_pool_pred_body(h_ref, oh_ref, w1_ref, b1_ref, w2_ref, b2_ref, out_ref):
    pooled = jax.lax.dot(oh_ref[...], h_ref[...],
                         precision=jax.lax.Precision.HIGHEST)
    t = jnp.maximum(_dot(pooled, w1_ref[...]) + b1_ref[...], 0.0)
    out_ref[...] = _dot(t, w2_ref[...]) + b2_ref[...]


def _tc_pool_pred(h, onehot, p1, p2):
    return pl.pallas_call(
        _tc_pool_pred_body,
        out_shape=jax.ShapeDtypeStruct((NG, 1), jnp.float32),
    )(h, onehot, p1["W"], p1["b"][None, :], p2["W"], p2["b"][None, :])


def _conv(h, q, k, v, dst, src, pskip, h0, pc_next):
    qd, ks = _sc_gather2(q, k, dst, src)
    logits = jnp.reshape(_tc_dot(qd, ks), (E,))
    mxp = _sc_max(logits, dst)
    ex, denp = _sc_conv_ex(logits, dst, mxp)
    outp = _sc_conv_scat(ex, denp, dst, src, v)
    return _tc_conv_fin(outp, h, pskip, h0, pc_next)


def kernel(x, pos, edge_index, node_subnode_index, subgraph_edge_index,
           subnode_node_index, batch, params):
    pos128 = jnp.pad(pos, ((0, 0), (0, C - 3)))

    e_dst, e_src = edge_index[1], edge_index[0]
    sets = [(ei[1], ei[0]) for ei in
            (node_subnode_index, subgraph_edge_index, subnode_node_index)]

    h = _tc_emb(x, params["emb_in"])
    pdr, psr = _sc_gather2(pos128, pos128, e_dst, e_src)
    dcol = _tc_dcol(pdr, psr)

    for lp in params["layers"]:
        h0 = h
        pe = lp["egnn"]
        hd, hs = _sc_gather2(h, h, e_dst, e_src)
        m = _tc_edge_mlp(hd, hs, dcol, pe)
        aggp = _sc_scatter_rows(m, e_dst)
        h, q, k, v = _tc_egnn_node(h, aggp, pe, lp["g2s"])
        (d1, s1), (d2, s2), (d3, s3) = sets
        h, q, k, v = _conv(h, q, k, v, d1, s1, lp["g2s"]["skip"], None, lp["sub"])
        h, q, k, v = _conv(h, q, k, v, d2, s2, lp["sub"]["skip"], None, lp["s2g"])
        h, _, _, _ = _conv(h, q, k, v, d3, s3, lp["s2g"]["skip"], h0, None)

    onehot = (batch[None, :] == jnp.arange(NG, dtype=batch.dtype)[:, None]).astype(jnp.float32)
    return _tc_pool_pred(h, onehot, params["pred1"], params["pred2"])
